# baseline (speedup 1.0000x reference)
import functools

import jax
import jax.numpy as jnp
from jax import lax
from jax.experimental import pallas as pl
from jax.experimental.pallas import tpu as pltpu
from jax.experimental.pallas import tpu_sc as plsc

N = 10000
E = 320000
D = 128

NC = 2
NS = 16
NW = NC * NS
C = 128
CHUNKS = 2 * (-(-E // (NW * C * 2)))
PAIRS = CHUNKS // 2
E_PAD = NW * C * CHUNKS
NP = 10240
RPT = NP // NS


def _sc_aggregate(z_hbm, src_hbm, dst_hbm, zeros_hbm, zeros_np_hbm,
                  ones_c_hbm, agg_out, deg_out, src_blk, dst0, dst1,
                  rows0, rows1, deg_v, ones_v, si0, si1, sg0, sg1, ss0,
                  ss1, sd0, sd1, agg_sh, deg_sh, *, compute_deg):
    cid = lax.axis_index("c")
    sid = lax.axis_index("s")
    wid = cid * NS + sid
    dst_idx = [dst0, dst1]
    rows = [rows0, rows1]
    sem_i = [si0, si1]
    sem_g = [sg0, sg1]
    sem_s = [ss0, ss1]
    sem_d = [sd0, sd1]

    pltpu.sync_copy(zeros_hbm, rows0)
    zd = [pltpu.async_copy(
        rows0, agg_sh.at[pl.ds((sid * (RPT // C) + k) * C, C)], ss0)
        for k in range(RPT // C)]
    for dd in zd:
        dd.wait()
    if compute_deg:
        pltpu.sync_copy(zeros_np_hbm.at[pl.ds(sid * RPT, RPT)], deg_v)
        pltpu.sync_copy(deg_v, deg_sh.at[pl.ds(sid * RPT, RPT)])
        pltpu.sync_copy(ones_c_hbm, ones_v)
    plsc.subcore_barrier()

    base_w = wid * (CHUNKS * C)
    pltpu.sync_copy(src_hbm.at[pl.ds(base_w, (CHUNKS + 1) * C)], src_blk)

    def issue(chunk, b):
        pltpu.async_copy(dst_hbm.at[pl.ds(base_w + chunk * C, C)],
                         dst_idx[b], sem_i[b])
        pltpu.async_copy(z_hbm.at[src_blk.at[pl.ds(chunk * C, C)]],
                         rows[b], sem_g[b])

    def scatter(b):
        s = pltpu.async_copy(rows[b], agg_sh.at[dst_idx[b]], sem_s[b],
                             add=True)
        d = None
        if compute_deg:
            d = pltpu.async_copy(ones_v, deg_sh.at[dst_idx[b]], sem_d[b],
                                 add=True)
        return s, d

    def drain_idx(b):
        pltpu.make_async_copy(src_hbm.at[pl.ds(0, C)], dst_idx[b],
                              sem_i[b]).wait()

    def drain_rows(b, sem):
        pltpu.make_async_copy(zeros_hbm, rows[b], sem).wait()

    def drain_deg(b):
        if compute_deg:
            pltpu.make_async_copy(ones_c_hbm, ones_v, sem_d[b]).wait()

    issue(0, 0)
    drain_idx(0)
    drain_rows(0, sem_g[0])
    issue(1, 1)
    s0, d0 = scatter(0)
    drain_idx(1)
    drain_rows(1, sem_g[1])
    s0.wait()
    if compute_deg:
        d0.wait()
    issue(2, 0)
    scatter(1)

    def body(p, carry):
        drain_idx(0)
        drain_rows(0, sem_g[0])
        scatter(0)
        drain_rows(1, sem_s[1])
        drain_deg(1)
        issue(2 * p + 1, 1)
        drain_idx(1)
        drain_rows(1, sem_g[1])
        drain_rows(0, sem_s[0])
        drain_deg(0)
        issue(2 * p + 2, 0)
        scatter(1)
        return carry

    lax.fori_loop(1, PAIRS, body, 0)
    drain_idx(0)
    drain_rows(0, sem_g[0])
    drain_rows(1, sem_s[1])
    drain_deg(1)
    plsc.subcore_barrier()

    outd = [None, None]
    for k in range(RPT // C):
        b = k % 2
        if outd[b] is not None:
            outd[b].wait()
        r = (sid * (RPT // C) + k) * C
        pltpu.async_copy(agg_sh.at[pl.ds(r, C)], rows[b], sem_g[b]).wait()
        outd[b] = pltpu.async_copy(rows[b], agg_out.at[cid, pl.ds(r, C)],
                                   sem_s[b])
    for dd in outd:
        if dd is not None:
            dd.wait()
    if compute_deg:
        pltpu.sync_copy(deg_sh.at[pl.ds(sid * RPT, RPT)], deg_v)
        pltpu.sync_copy(deg_v, deg_out.at[cid, pl.ds(sid * RPT, RPT)])


@functools.lru_cache(maxsize=None)
def _make_sc_pass(compute_deg):
    mesh = plsc.VectorSubcoreMesh(core_axis_name="c", subcore_axis_name="s",
                                  num_cores=NC, num_subcores=NS)
    out_type = [jax.ShapeDtypeStruct((NC, NP, D), jnp.float32)]
    scratch = [
        pltpu.VMEM(((CHUNKS + 1) * C,), jnp.int32),
        pltpu.VMEM((C,), jnp.int32),
        pltpu.VMEM((C,), jnp.int32),
        pltpu.VMEM((C, D), jnp.float32),
        pltpu.VMEM((C, D), jnp.float32),
        pltpu.VMEM((RPT,), jnp.float32),
        pltpu.VMEM((C,), jnp.float32),
        pltpu.SemaphoreType.DMA,
        pltpu.SemaphoreType.DMA,
        pltpu.SemaphoreType.DMA,
        pltpu.SemaphoreType.DMA,
        pltpu.SemaphoreType.DMA,
        pltpu.SemaphoreType.DMA,
        pltpu.SemaphoreType.DMA,
        pltpu.SemaphoreType.DMA,
        pltpu.VMEM_SHARED((NP, D), jnp.float32),
        pltpu.VMEM_SHARED((NP,), jnp.float32),
    ]
    if compute_deg:
        out_type.append(jax.ShapeDtypeStruct((NC, NP), jnp.float32))
        body = functools.partial(_sc_aggregate, compute_deg=True)
    else:
        def body(z, s, d, z0, znp, o1, agg_out, *rest):
            _sc_aggregate(z, s, d, z0, znp, o1, agg_out, None, *rest,
                          compute_deg=False)
    return pl.kernel(body, out_type=out_type, mesh=mesh,
                     scratch_types=scratch)


def _tc_layer_body(agg_ref, deg_ref, z_ref, wl_ref, wr_ref, b_ref, out_ref):
    a = agg_ref[0] + agg_ref[1]
    dg = deg_ref[0] + deg_ref[1]
    inv = 1.0 / jnp.maximum(dg, 1.0)
    mean = a * inv[:, None]
    out = (jnp.dot(mean, wl_ref[...], preferred_element_type=jnp.float32)
           + jnp.dot(z_ref[...], wr_ref[...],
                     preferred_element_type=jnp.float32)
           + b_ref[...])
    out_ref[...] = jnp.maximum(out, 0.0)


BR = 2048


def _tc_layer(agg, deg, z, W_l, W_r, b):
    grid = (NP // BR,)
    return pl.pallas_call(
        _tc_layer_body,
        grid=grid,
        in_specs=[
            pl.BlockSpec((NC, BR, D), lambda i: (0, i, 0)),
            pl.BlockSpec((NC, BR), lambda i: (0, i)),
            pl.BlockSpec((BR, D), lambda i: (i, 0)),
            pl.BlockSpec((D, D), lambda i: (0, 0)),
            pl.BlockSpec((D, D), lambda i: (0, 0)),
            pl.BlockSpec((1, D), lambda i: (0, 0)),
        ],
        out_specs=pl.BlockSpec((BR, D), lambda i: (i, 0)),
        out_shape=jax.ShapeDtypeStruct((NP, D), jnp.float32),
    )(agg, deg, z, W_l, W_r, b)


@jax.jit
def kernel(x, edge_index, W_l1, b_l1, W_r1, W_l2, b_l2, W_r2):
    src = edge_index[0].astype(jnp.int32)
    dst = edge_index[1].astype(jnp.int32)
    src_p = jnp.concatenate(
        [src, jnp.zeros((E_PAD + C - E,), jnp.int32)])
    dst_p = jnp.concatenate(
        [dst, jnp.full((E_PAD + C - E,), N, jnp.int32)])
    x_p = jnp.pad(x, ((0, NP - N), (0, 0)))
    zeros = jnp.zeros((C, D), jnp.float32)
    zeros_np = jnp.zeros((NP,), jnp.float32)
    ones_c = jnp.ones((C,), jnp.float32)

    agg1, deg = _make_sc_pass(True)(x_p, src_p, dst_p, zeros, zeros_np,
                                    ones_c)
    h1 = _tc_layer(agg1, deg, x_p, W_l1, W_r1, b_l1.reshape(1, D))
    (agg2,) = _make_sc_pass(False)(h1, src_p, dst_p, zeros, zeros_np, ones_c)
    h2 = _tc_layer(agg2, deg, h1, W_l2, W_r2, b_l2.reshape(1, D))
    return h2[:N]
